# baseline (speedup 1.0000x reference)
import functools

import jax
import jax.numpy as jnp
from jax import lax
from jax.experimental import pallas as pl
from jax.experimental.pallas import tpu as pltpu
from jax.experimental.pallas import tpu_sc as plsc

_B, _S, _D, _V = 1024, 50, 512, 100000
_NC, _NS = 2, 16
_NW = _NC * _NS
_BPW = _B // _NW
_NCHUNK = _BPW
_CHUNK = _S
_LANES = 16


def _pe_table():
    i = jnp.arange(_S, dtype=jnp.float32)[:, None]
    j = jnp.arange(_D // 2, dtype=jnp.float32)[None, :]
    ang = i / jnp.power(jnp.float32(10000.0), 2.0 * j / _D)
    pe = jnp.zeros((_S, _D), dtype=jnp.float32)
    pe = pe.at[:, 0::2].set(jnp.sin(ang))
    pe = pe.at[:, 1::2].set(jnp.cos(ang))
    return pe


_mesh = plsc.VectorSubcoreMesh(core_axis_name="c", subcore_axis_name="s")


@functools.partial(
    pl.kernel,
    mesh=_mesh,
    out_type=jax.ShapeDtypeStruct((_B, _S, _D), jnp.float32),
    scratch_types=[
        pltpu.VMEM((_NCHUNK, _CHUNK), jnp.int32),
        pltpu.VMEM((_S, _D), jnp.float32),
        pltpu.VMEM((3, _CHUNK, _D), jnp.float32),
        pltpu.SemaphoreType.DMA,
        pltpu.SemaphoreType.DMA,
        pltpu.SemaphoreType.DMA,
        pltpu.SemaphoreType.DMA,
        pltpu.SemaphoreType.DMA,
        pltpu.SemaphoreType.DMA,
    ],
    compiler_params=pltpu.CompilerParams(use_tc_tiling_on_sc=False),
)
def _sc_lookup(x_hbm, pe_hbm, tbl_hbm, out_hbm, idx_v, pe_v, rows_v,
               gsem0, gsem1, gsem2, ssem0, ssem1, ssem2):
    gsems = (gsem0, gsem1, gsem2)
    ssems = (ssem0, ssem1, ssem2)
    wid = lax.axis_index("s") * _NC + lax.axis_index("c")
    pltpu.sync_copy(x_hbm.at[wid], idx_v)
    pltpu.sync_copy(pe_hbm, pe_v)

    def add_pe(i, buf):
        vals = []
        for c in range(_D // _LANES):
            sl = pl.ds(c * _LANES, _LANES)
            vals.append((sl, pe_v[i, sl]))
        for sl, v in vals:
            plsc.addupdate(rows_v.at[buf, i, sl], v)
        return buf

    def issue_gather(j, b):
        pltpu.async_copy(tbl_hbm.at[idx_v.at[j]], rows_v.at[b], gsems[b])

    def drain_gather(b):
        pltpu.make_async_copy(
            tbl_hbm.at[pl.ds(0, _CHUNK)], rows_v.at[b], gsems[b]).wait()

    def drain_scatter(b):
        pltpu.make_async_copy(rows_v.at[b], out_hbm.at[0], ssems[b]).wait()

    issue_gather(0, 0)
    issue_gather(1, 1)

    def tri_body(j3, _):
        for sub in range(3):
            j = 3 * j3 + sub
            b = sub
            nb = (sub + 2) % 3
            drain_gather(b)

            @pl.when(j >= 1)
            def _wait_prev_scatter():
                drain_scatter(nb)

            issue_gather(j + 2, nb)
            lax.fori_loop(0, _CHUNK, add_pe, b)
            pltpu.async_copy(rows_v.at[b], out_hbm.at[wid * _NCHUNK + j],
                             ssems[b])
        return _

    lax.fori_loop(0, _NCHUNK // 3 - 1, tri_body, 0)

    for j in (27, 28, 29, 30, 31):
        b = j % 3
        drain_gather(b)
        drain_scatter((b + 2) % 3)
        if j + 2 < _NCHUNK:
            issue_gather(j + 2, (b + 2) % 3)
        lax.fori_loop(0, _CHUNK, add_pe, b)
        pltpu.async_copy(rows_v.at[b], out_hbm.at[wid * _NCHUNK + j],
                         ssems[b])
    drain_scatter(31 % 3)


def kernel(x, offsets, table):
    del offsets
    x3 = x.reshape(_NW, _NCHUNK, _CHUNK)
    return _sc_lookup(x3, _pe_table(), table)
